# baseline (speedup 1.0000x reference)
import functools

import jax
import jax.numpy as jnp
from jax import lax
from jax.experimental import pallas as pl
from jax.experimental.pallas import tpu as pltpu
from jax.experimental.pallas import tpu_sc as plsc

_CHUNK = 256


def _gather_kernel(B, D, b_per_w, num_cores):
    C = _CHUNK
    nchunk = 2 * b_per_w // C
    half = nchunk // 2
    mesh = plsc.VectorSubcoreMesh(core_axis_name="c", subcore_axis_name="s")

    @functools.partial(
        pl.kernel,
        mesh=mesh,
        out_type=(
            jax.ShapeDtypeStruct((B, D), jnp.float32),
            jax.ShapeDtypeStruct((B, D), jnp.float32),
        ),
        scratch_types=[
            pltpu.VMEM((2 * b_per_w,), jnp.int32),
            pltpu.VMEM((C, D), jnp.float32),
            pltpu.VMEM((C, D), jnp.float32),
            pltpu.SemaphoreType.DMA,
            pltpu.SemaphoreType.DMA,
            pltpu.SemaphoreType.DMA,
            pltpu.SemaphoreType.DMA,
        ],
    )
    def k(x_hbm, y_hbm, emb_hbm, zx_hbm, zy_hbm, idx_v, buf0, buf1, g0, g1, s0, s1):
        wid = lax.axis_index("s") * num_cores + lax.axis_index("c")
        base = wid * b_per_w
        pltpu.sync_copy(x_hbm.at[pl.ds(base, b_per_w)], idx_v.at[pl.ds(0, b_per_w)])
        pltpu.sync_copy(
            y_hbm.at[pl.ds(base, b_per_w)], idx_v.at[pl.ds(b_per_w, b_per_w)]
        )
        bufs = (buf0, buf1)
        gsems = (g0, g1)
        ssems = (s0, s1)

        def start_gather(c):
            return pltpu.async_copy(
                emb_hbm.at[idx_v.at[pl.ds(c * C, C)]], bufs[c % 2], gsems[c % 2]
            )

        def out_slice(c):
            out = zx_hbm if c < half else zy_hbm
            return out.at[pl.ds(base + (c % half) * C, C)]

        gathers = [None] * nchunk
        stores = [None] * nchunk
        for c in range(min(2, nchunk)):
            gathers[c] = start_gather(c)
        for c in range(nchunk):
            gathers[c].wait()
            stores[c] = pltpu.async_copy(bufs[c % 2], out_slice(c), ssems[c % 2])
            if c + 2 < nchunk:
                stores[c].wait()
                gathers[c + 2] = start_gather(c + 2)
        for c in range(max(0, nchunk - 2), nchunk):
            stores[c].wait()

    return k


def kernel(x, y, action, emb):
    (B,) = x.shape
    V, D = emb.shape
    info = plsc.get_sparse_core_info()
    nw = info.num_cores * info.num_subcores
    b_per_w = B // nw
    k = _gather_kernel(B, D, b_per_w, info.num_cores)
    zx, zy = k(x.astype(jnp.int32), y.astype(jnp.int32), emb)
    return (zx, zy, action)
